# baseline (speedup 1.0000x reference)
import functools

import jax
import jax.numpy as jnp
from jax import lax
from jax.experimental import pallas as pl
from jax.experimental.pallas import tpu as pltpu
from jax.experimental.pallas import tpu_sc as plsc

_B = 16384
_E = 64
_NUM_NEG = 5
_VOCAB = 1000000
_NJ = _NUM_NEG + 1
_NW = 32
_BPW = _B // _NW
_CHUNK = 64
_NCHUNK = _BPW // _CHUNK
_LANES = 16

_mesh = plsc.VectorSubcoreMesh(core_axis_name="c", subcore_axis_name="s")


@functools.partial(
    pl.kernel,
    mesh=_mesh,
    out_type=jax.ShapeDtypeStruct((_NJ, _B), jnp.float32),
    scratch_types=[
        pltpu.VMEM((_E, _BPW), jnp.float32),
        pltpu.VMEM((_NJ, _BPW), jnp.int32),
        pltpu.VMEM((_NJ, _CHUNK, _E), jnp.float32),
        pltpu.VMEM((_NJ, _BPW), jnp.float32),
        pltpu.SemaphoreType.DMA,
    ],
    compiler_params=pltpu.CompilerParams(
        needs_layout_passes=False, use_tc_tiling_on_sc=False
    ),
)
def _sc_dots(hT_hbm, idx_hbm, table_hbm, out_hbm, hT_v, idx_v, buf_v, dots_v, sem):
    wid = lax.axis_index("s") * 2 + lax.axis_index("c")
    base = wid * _BPW
    pltpu.sync_copy(hT_hbm.at[:, pl.ds(base, _BPW)], hT_v)
    pltpu.sync_copy(idx_hbm.at[:, pl.ds(base, _BPW)], idx_v)

    lane_iota = lax.iota(jnp.int32, _LANES)

    for c in range(_NCHUNK):
        cps = [
            pltpu.async_copy(
                table_hbm.at[idx_v.at[j, pl.ds(c * _CHUNK, _CHUNK)]],
                buf_v.at[j],
                sem,
            )
            for j in range(_NJ)
        ]
        for cp in cps:
            cp.wait()
        for g in range(_CHUNK // _LANES):
            off = c * _CHUNK + g * _LANES
            row_idx = lane_iota + (g * _LANES)
            zero = jnp.zeros((_LANES,), jnp.float32)

            def dbody(dd, carry):
                accs = carry
                h = hT_v[dd, pl.ds(off, _LANES)]
                col = jnp.full((_LANES,), dd, jnp.int32)
                new = []
                for j in range(_NJ):
                    e = plsc.load_gather(buf_v.at[j], [row_idx, col])
                    new.append(accs[j] + e * h)
                return tuple(new)

            accs = lax.fori_loop(0, _E, dbody, (zero,) * _NJ)
            for j in range(_NJ):
                dots_v[j, pl.ds(off, _LANES)] = accs[j]

    pltpu.sync_copy(dots_v, out_hbm.at[:, pl.ds(base, _BPW)])


def _tc_loss_body(d_ref, o_ref):
    d = d_ref[...]
    rows = lax.broadcasted_iota(jnp.int32, d.shape, 0)
    z = jnp.where(rows == 0, d, -d)
    logsig = -jnp.log(1.0 + jnp.exp(-z))
    o_ref[0, 0] = -jnp.sum(logsig) * (1.0 / _B)


def _tc_loss(dots):
    return pl.pallas_call(
        _tc_loss_body,
        out_shape=jax.ShapeDtypeStruct((1, 1), jnp.float32),
        in_specs=[pl.BlockSpec(memory_space=pltpu.VMEM)],
        out_specs=pl.BlockSpec(memory_space=pltpu.SMEM),
    )(dots)


def kernel(hidden, positives, table):
    negatives = jax.random.randint(
        jax.random.key(42), (_B, _NUM_NEG), 1, _VOCAB - 1, dtype=jnp.int32
    )
    idx = jnp.concatenate([positives[None, :], negatives.T], axis=0)
    hT = hidden.T
    dots = _sc_dots(hT, idx, table)
    return _tc_loss(dots)[0, 0]
